# baseline (speedup 1.0000x reference)
import functools

import jax
import jax.numpy as jnp
from jax import lax
from jax.experimental import pallas as pl
from jax.experimental.pallas import tpu as pltpu
from jax.experimental.pallas import tpu_sc as plsc

D_MODEL = 64
VOCAB = 1000000
SCALE = 8.0

S_DIM = 4096
T_DIM = 200
NC, NS = 2, 16
NW = NC * NS
LANES = 16

VBLK = 128
N_VFULL = VOCAB // VBLK
V_TAIL = VOCAB - N_VFULL * VBLK
A_EXTRA = N_VFULL % NW
A_BASE = N_VFULL // NW

SBLK = S_DIM // NW


def _iota16():
    return lax.iota(jnp.int32, 16)


def _conv_kernel(wt_hbm, tail_hbm, w128_hbm, in_v, out_v, gsem, osem):
    wid = lax.axis_index("s") * NC + lax.axis_index("c")
    n_blk = A_BASE + jnp.where(wid < A_EXTRA, 1, 0)

    def blk_idx(k):
        return k * NW + wid

    def start_in(k, b):
        pltpu.async_copy(
            wt_hbm.at[:, pl.ds(blk_idx(k) * VBLK, VBLK)], in_v.at[b], gsem.at[b]
        )

    def wait_in(k, b):
        pltpu.make_async_copy(
            wt_hbm.at[:, pl.ds(blk_idx(k) * VBLK, VBLK)], in_v.at[b], gsem.at[b]
        ).wait()

    def start_out(k, b):
        pltpu.async_copy(
            out_v.at[b], w128_hbm.at[pl.ds(blk_idx(k) * (VBLK // 2), VBLK // 2)],
            osem.at[b],
        )

    def wait_out(k, b):
        pltpu.make_async_copy(
            out_v.at[b], w128_hbm.at[pl.ds(blk_idx(k) * (VBLK // 2), VBLK // 2)],
            osem.at[b],
        ).wait()

    lane = _iota16()
    row_idx = [lane + (c0 % D_MODEL) for c0 in range(0, 2 * D_MODEL, LANES)]

    def transpose_blk(b):
        def qloop(q4, carry):
            for u in range(4):
                q = q4 * 4 + u
                for half in range(2):
                    col = jnp.full((16,), 2 * q + half, jnp.int32)
                    for c4 in range(4):
                        c0 = half * D_MODEL + c4 * LANES
                        v = plsc.load_gather(in_v.at[b], [row_idx[c0 // LANES], col])
                        out_v[b, q, pl.ds(c0, LANES)] = v
            return carry

        lax.fori_loop(0, LANES, qloop, 0, unroll=False)

    start_in(0, 0)

    def step(k, b):
        @pl.when(k + 1 < n_blk)
        def _():
            start_in(k + 1, 1 - b)

        wait_in(k, b)

        @pl.when(k >= 2)
        def _():
            wait_out(k - 2, b)

        transpose_blk(b)
        start_out(k, b)

    def pair_body(g, carry):
        step(2 * g, 0)
        step(2 * g + 1, 1)
        return carry

    lax.fori_loop(0, n_blk // 2, pair_body, 0, unroll=False)

    @pl.when(n_blk % 2 == 1)
    def _():
        step(n_blk - 1, 0)

    wait_out(n_blk - 2, 0)
    wait_out(n_blk - 1, 1)

    @pl.when(wid == A_EXTRA)
    def _():
        pltpu.async_copy(tail_hbm, in_v.at[0], gsem.at[0]).wait()

        def qloop_t(q, carry):
            for half in range(2):
                col = jnp.full((16,), 2 * q + half, jnp.int32)
                for c4 in range(4):
                    c0 = half * D_MODEL + c4 * LANES
                    v = plsc.load_gather(in_v.at[0], [row_idx[c0 // LANES], col])
                    out_v[0, q, pl.ds(c0, LANES)] = v
            return carry

        lax.fori_loop(0, V_TAIL // 2, qloop_t, 0, unroll=False)
        pltpu.async_copy(
            out_v.at[0, pl.ds(0, V_TAIL // 2)],
            w128_hbm.at[pl.ds(N_VFULL * (VBLK // 2), V_TAIL // 2)],
            osem.at[0],
        ).wait()


def _gather_kernel(xt_hbm, w128_hbm, out_hbm, idx_v, pidx_v, rows_v, ot_v,
                   gsem, osem):
    wid = lax.axis_index("s") * NC + lax.axis_index("c")
    s0 = wid * SBLK
    pltpu.sync_copy(xt_hbm.at[:, pl.ds(s0, SBLK)], idx_v)

    def pidx_body(t, carry):
        for l in range(SBLK // LANES):
            sl = pl.ds(l * LANES, LANES)
            pidx_v[t, sl] = jax.lax.shift_right_logical(idx_v[t, sl], 1)
        return carry

    lax.fori_loop(0, T_DIM, pidx_body, 0, unroll=False)

    def start_gather(t, b):
        pltpu.async_copy(w128_hbm.at[pidx_v.at[t]], rows_v.at[b], gsem.at[b])

    def wait_gather(t, b):
        pltpu.make_async_copy(
            w128_hbm.at[pidx_v.at[t]], rows_v.at[b], gsem.at[b]
        ).wait()

    def start_out(t, b):
        pltpu.async_copy(
            ot_v.at[b], out_hbm.at[t, :, pl.ds(s0, SBLK)], osem.at[b]
        )

    def wait_out(t, b):
        pltpu.make_async_copy(
            ot_v.at[b], out_hbm.at[t, :, pl.ds(s0, SBLK)], osem.at[b]
        ).wait()

    lane = _iota16()
    row_idx = [lane + l * LANES for l in range(SBLK // LANES)]

    def transpose_blk(t, b):
        hvecs = []
        for l in range(SBLK // LANES):
            sl = pl.ds(l * LANES, LANES)
            hvecs.append(
                jax.lax.shift_left(jnp.bitwise_and(idx_v[t, sl], 1), 6)
            )

        def jloop(j4, carry):
            for u in range(4):
                j = j4 * 4 + u
                for l in range(SBLK // LANES):
                    col = hvecs[l] + j
                    v = plsc.load_gather(rows_v.at[b], [row_idx[l], col])
                    ot_v[b, j, pl.ds(l * LANES, LANES)] = v * SCALE
            return carry

        lax.fori_loop(0, LANES, jloop, 0, unroll=False)

    start_gather(0, 0)

    def step(t, b):
        @pl.when(t + 1 < T_DIM)
        def _():
            start_gather(t + 1, 1 - b)

        wait_gather(t, b)

        @pl.when(t >= 2)
        def _():
            wait_out(t - 2, b)

        transpose_blk(t, b)
        start_out(t, b)

    def pair_body(g, carry):
        step(2 * g, 0)
        step(2 * g + 1, 1)
        return carry

    lax.fori_loop(0, T_DIM // 2, pair_body, 0, unroll=False)
    wait_out(T_DIM - 2, 0)
    wait_out(T_DIM - 1, 1)


@jax.jit
def _emb(xt, wt):
    mesh = plsc.VectorSubcoreMesh(core_axis_name="c", subcore_axis_name="s")
    conv = functools.partial(
        pl.kernel,
        mesh=mesh,
        out_type=jax.ShapeDtypeStruct((VOCAB // 2, 2 * D_MODEL), jnp.float32),
        scratch_types=[
            pltpu.VMEM((2, D_MODEL, VBLK), jnp.float32),
            pltpu.VMEM((2, VBLK // 2, 2 * D_MODEL), jnp.float32),
            pltpu.SemaphoreType.DMA((2,)),
            pltpu.SemaphoreType.DMA((2,)),
        ],
        compiler_params=pltpu.CompilerParams(needs_layout_passes=False),
    )(_conv_kernel)
    tail_w = jnp.pad(wt[:, N_VFULL * VBLK:], ((0, 0), (0, VBLK - V_TAIL)))
    w128 = conv(wt, tail_w)

    gath = functools.partial(
        pl.kernel,
        mesh=mesh,
        out_type=jax.ShapeDtypeStruct((T_DIM, D_MODEL, S_DIM), jnp.float32),
        scratch_types=[
            pltpu.VMEM((T_DIM, SBLK), jnp.int32),
            pltpu.VMEM((T_DIM, SBLK), jnp.int32),
            pltpu.VMEM((2, SBLK, 2 * D_MODEL), jnp.float32),
            pltpu.VMEM((2, D_MODEL, SBLK), jnp.float32),
            pltpu.SemaphoreType.DMA((2,)),
            pltpu.SemaphoreType.DMA((2,)),
        ],
        compiler_params=pltpu.CompilerParams(needs_layout_passes=False),
    )(_gather_kernel)
    return gath(xt, w128)


def kernel(x, emb_weight):
    out3 = _emb(x.T, emb_weight.T)
    return out3.transpose(2, 0, 1)


# speedup vs baseline: 1.9682x; 1.9682x over previous
import functools

import jax
import jax.numpy as jnp
from jax import lax
from jax.experimental import pallas as pl
from jax.experimental.pallas import tpu as pltpu
from jax.experimental.pallas import tpu_sc as plsc

D_MODEL = 64
VOCAB = 1000000
SCALE = 8.0

S_DIM = 4096
T_DIM = 200
NC, NS = 2, 16
NW = NC * NS
LANES = 16

VBLK = 128
N_VFULL = VOCAB // VBLK
V_TAIL = VOCAB - N_VFULL * VBLK
A_EXTRA = N_VFULL % NW
A_BASE = N_VFULL // NW

SBLK = S_DIM // NW


def _iota16():
    return lax.iota(jnp.int32, 16)


def _conv_kernel(wt_hbm, tail_hbm, w128_hbm, in_v, out_v, gsem, osem):
    wid = lax.axis_index("s") * NC + lax.axis_index("c")
    n_blk = A_BASE + jnp.where(wid < A_EXTRA, 1, 0)

    def blk_idx(k):
        return k * NW + wid

    def start_in(k, b):
        pltpu.async_copy(
            wt_hbm.at[:, pl.ds(blk_idx(k) * VBLK, VBLK)], in_v.at[b], gsem.at[b]
        )

    def wait_in(k, b):
        pltpu.make_async_copy(
            wt_hbm.at[:, pl.ds(blk_idx(k) * VBLK, VBLK)], in_v.at[b], gsem.at[b]
        ).wait()

    def start_out(k, b):
        pltpu.async_copy(
            out_v.at[b], w128_hbm.at[pl.ds(blk_idx(k) * (VBLK // 2), VBLK // 2)],
            osem.at[b],
        )

    def wait_out(k, b):
        pltpu.make_async_copy(
            out_v.at[b], w128_hbm.at[pl.ds(blk_idx(k) * (VBLK // 2), VBLK // 2)],
            osem.at[b],
        ).wait()

    lane = _iota16()
    row_idx = [lane + (c0 % D_MODEL) for c0 in range(0, 2 * D_MODEL, LANES)]

    def transpose_blk(b):
        @plsc.parallel_loop(0, VBLK // 2, 1, unroll=8)
        def _(q):
            for half in range(2):
                col = jnp.full((16,), 2 * q + half, jnp.int32)
                for c4 in range(4):
                    c0 = half * D_MODEL + c4 * LANES
                    v = plsc.load_gather(in_v.at[b], [row_idx[c0 // LANES], col])
                    out_v[b, q, pl.ds(c0, LANES)] = v

    start_in(0, 0)

    def step(k, b):
        @pl.when(k + 1 < n_blk)
        def _():
            start_in(k + 1, 1 - b)

        wait_in(k, b)

        @pl.when(k >= 2)
        def _():
            wait_out(k - 2, b)

        transpose_blk(b)
        start_out(k, b)

    def pair_body(g, carry):
        step(2 * g, 0)
        step(2 * g + 1, 1)
        return carry

    lax.fori_loop(0, n_blk // 2, pair_body, 0, unroll=False)

    @pl.when(n_blk % 2 == 1)
    def _():
        step(n_blk - 1, 0)

    wait_out(n_blk - 2, 0)
    wait_out(n_blk - 1, 1)

    @pl.when(wid == A_EXTRA)
    def _():
        pltpu.async_copy(tail_hbm, in_v.at[0], gsem.at[0]).wait()

        @plsc.parallel_loop(0, V_TAIL // 2, 1, unroll=8)
        def _(q):
            for half in range(2):
                col = jnp.full((16,), 2 * q + half, jnp.int32)
                for c4 in range(4):
                    c0 = half * D_MODEL + c4 * LANES
                    v = plsc.load_gather(in_v.at[0], [row_idx[c0 // LANES], col])
                    out_v[0, q, pl.ds(c0, LANES)] = v
        pltpu.async_copy(
            out_v.at[0, pl.ds(0, V_TAIL // 2)],
            w128_hbm.at[pl.ds(N_VFULL * (VBLK // 2), V_TAIL // 2)],
            osem.at[0],
        ).wait()


def _gather_kernel(xt_hbm, w128_hbm, out_hbm, idx_v, pidx_v, rows_v, ot_v,
                   gsem, osem):
    wid = lax.axis_index("s") * NC + lax.axis_index("c")
    s0 = wid * SBLK
    pltpu.sync_copy(xt_hbm.at[:, pl.ds(s0, SBLK)], idx_v)

    def pidx_body(t, carry):
        for l in range(SBLK // LANES):
            sl = pl.ds(l * LANES, LANES)
            pidx_v[t, sl] = jax.lax.shift_right_logical(idx_v[t, sl], 1)
        return carry

    lax.fori_loop(0, T_DIM, pidx_body, 0, unroll=False)

    def start_gather(t, b):
        pltpu.async_copy(w128_hbm.at[pidx_v.at[t]], rows_v.at[b], gsem.at[b])

    def wait_gather(t, b):
        pltpu.make_async_copy(
            w128_hbm.at[pidx_v.at[t]], rows_v.at[b], gsem.at[b]
        ).wait()

    def start_out(t, b):
        pltpu.async_copy(
            ot_v.at[b], out_hbm.at[t, :, pl.ds(s0, SBLK)], osem.at[b]
        )

    def wait_out(t, b):
        pltpu.make_async_copy(
            ot_v.at[b], out_hbm.at[t, :, pl.ds(s0, SBLK)], osem.at[b]
        ).wait()

    lane = _iota16()
    row_idx = [lane + l * LANES for l in range(SBLK // LANES)]

    def transpose_blk(t, b):
        hvecs = []
        for l in range(SBLK // LANES):
            sl = pl.ds(l * LANES, LANES)
            hvecs.append(
                jax.lax.shift_left(jnp.bitwise_and(idx_v[t, sl], 1), 6)
            )

        @plsc.parallel_loop(0, D_MODEL, 1, unroll=8)
        def _(j):
            for l in range(SBLK // LANES):
                col = hvecs[l] + j
                v = plsc.load_gather(rows_v.at[b], [row_idx[l], col])
                ot_v[b, j, pl.ds(l * LANES, LANES)] = v * SCALE

    start_gather(0, 0)

    def step(t, b):
        @pl.when(t + 1 < T_DIM)
        def _():
            start_gather(t + 1, 1 - b)

        wait_gather(t, b)

        @pl.when(t >= 2)
        def _():
            wait_out(t - 2, b)

        transpose_blk(t, b)
        start_out(t, b)

    def pair_body(g, carry):
        step(2 * g, 0)
        step(2 * g + 1, 1)
        return carry

    lax.fori_loop(0, T_DIM // 2, pair_body, 0, unroll=False)
    wait_out(T_DIM - 2, 0)
    wait_out(T_DIM - 1, 1)


@jax.jit
def _emb(xt, wt):
    mesh = plsc.VectorSubcoreMesh(core_axis_name="c", subcore_axis_name="s")
    conv = functools.partial(
        pl.kernel,
        mesh=mesh,
        out_type=jax.ShapeDtypeStruct((VOCAB // 2, 2 * D_MODEL), jnp.float32),
        scratch_types=[
            pltpu.VMEM((2, D_MODEL, VBLK), jnp.float32),
            pltpu.VMEM((2, VBLK // 2, 2 * D_MODEL), jnp.float32),
            pltpu.SemaphoreType.DMA((2,)),
            pltpu.SemaphoreType.DMA((2,)),
        ],
        compiler_params=pltpu.CompilerParams(needs_layout_passes=False),
    )(_conv_kernel)
    tail_w = jnp.pad(wt[:, N_VFULL * VBLK:], ((0, 0), (0, VBLK - V_TAIL)))
    w128 = conv(wt, tail_w)

    gath = functools.partial(
        pl.kernel,
        mesh=mesh,
        out_type=jax.ShapeDtypeStruct((T_DIM, D_MODEL, S_DIM), jnp.float32),
        scratch_types=[
            pltpu.VMEM((T_DIM, SBLK), jnp.int32),
            pltpu.VMEM((T_DIM, SBLK), jnp.int32),
            pltpu.VMEM((2, SBLK, 2 * D_MODEL), jnp.float32),
            pltpu.VMEM((2, D_MODEL, SBLK), jnp.float32),
            pltpu.SemaphoreType.DMA((2,)),
            pltpu.SemaphoreType.DMA((2,)),
        ],
        compiler_params=pltpu.CompilerParams(needs_layout_passes=False),
    )(_gather_kernel)
    return gath(xt, w128)


def kernel(x, emb_weight):
    out3 = _emb(x.T, emb_weight.T)
    return out3.transpose(2, 0, 1)
